# Initial kernel scaffold; baseline (speedup 1.0000x reference)
import jax
import jax.numpy as jnp
from jax.experimental import pallas as pl


def kernel(token_ids, emb_weight, proj_weight, scale):
    raise NotImplementedError("write your pallas kernel here")


# speedup vs baseline: 16.8521x; 16.8521x over previous
import functools

import jax
import jax.numpy as jnp
from jax import lax
from jax.experimental import pallas as pl
from jax.experimental.pallas import tpu as pltpu
from jax.experimental.pallas import tpu_sc as plsc

BUCKETS = 1000000
MOD = BUCKETS - 1
EDIM = 32
MDIM = 128
B = 4096
SEQ = 200
TOTAL = B * SEQ

LANES = 16
NW = 32
PER_W = TOTAL // NW
CHUNK = 2560
NCHUNK = PER_W // CHUNK
GSZ = 128
NG = CHUNK // GSZ


def _sc_hash_gather():
    mesh = plsc.VectorSubcoreMesh(core_axis_name="c", subcore_axis_name="s")

    @functools.partial(
        pl.kernel,
        mesh=mesh,
        compiler_params=pltpu.CompilerParams(use_tc_tiling_on_sc=False),
        out_type=jax.ShapeDtypeStruct((TOTAL, EDIM), jnp.float32),
        scratch_types=[
            pltpu.VMEM((CHUNK,), jnp.int32),
            pltpu.VMEM((CHUNK,), jnp.int32),
            pltpu.VMEM((CHUNK,), jnp.int32),
            pltpu.VMEM((CHUNK, EDIM), jnp.float32),
            pltpu.SemaphoreType.DMA,
        ],
    )
    def body(cur_hbm, prev_hbm, emb_hbm, out_hbm, cur_v, prev_v, idx_v, rows_v, sem):
        wid = lax.axis_index("s") * 2 + lax.axis_index("c")
        wbase = wid * PER_W

        def chunk_body(s, carry):
            base = wbase + s * CHUNK
            pltpu.sync_copy(cur_hbm.at[pl.ds(base, CHUNK)], cur_v)
            pltpu.sync_copy(prev_hbm.at[pl.ds(base, CHUNK)], prev_v)

            def hash_body(i, c2):
                off = i * LANES
                t1 = cur_v[pl.ds(off, LANES)]
                t0 = prev_v[pl.ds(off, LANES)]
                a = t1 * jnp.int32(36313)
                bb = t0 * jnp.int32(27191)
                r = lax.rem(lax.bitwise_xor(a, bb), jnp.int32(MOD))
                r = jnp.where(r < 0, r + jnp.int32(MOD), r)
                p = base + off + lax.iota(jnp.int32, LANES)
                first = lax.rem(p, jnp.int32(SEQ)) == 0
                idx_v[pl.ds(off, LANES)] = jnp.where(first, jnp.int32(MOD), r)
                return c2

            lax.fori_loop(0, CHUNK // LANES, hash_body, 0)

            copies = []
            for g in range(NG):
                copies.append(
                    pltpu.async_copy(
                        emb_hbm.at[idx_v.at[pl.ds(g * GSZ, GSZ)]],
                        rows_v.at[pl.ds(g * GSZ, GSZ)],
                        sem,
                    )
                )
            for cp in copies:
                cp.wait()
            pltpu.sync_copy(rows_v, out_hbm.at[pl.ds(base, CHUNK)])
            return carry

        lax.fori_loop(0, NCHUNK, chunk_body, 0)

    return body


_GATHER = _sc_hash_gather()

PACK = MDIM // EDIM
ROWS_BLK = 2048


def _proj_body(h_ref, p_ref, o_ref):
    o_ref[...] = lax.dot_general(
        h_ref[...], p_ref[...], (((1,), (0,)), ((), ())),
        preferred_element_type=jnp.float32,
    )


def _project(h_packed, p4):
    n = TOTAL // PACK
    grid = (n // ROWS_BLK,)
    return pl.pallas_call(
        _proj_body,
        grid=grid,
        in_specs=[
            pl.BlockSpec((ROWS_BLK, MDIM), lambda i: (i, 0)),
            pl.BlockSpec((MDIM, PACK * MDIM), lambda i: (0, 0)),
        ],
        out_specs=pl.BlockSpec((ROWS_BLK, PACK * MDIM), lambda i: (i, 0)),
        out_shape=jax.ShapeDtypeStruct((n, PACK * MDIM), jnp.float32),
    )(h_packed, p4)


def kernel(token_ids, emb_weight, proj_weight, scale):
    tok = token_ids.reshape(-1).astype(jnp.int32)
    prev = jnp.concatenate([tok[:1], tok[:-1]])

    emb_packed = lax.optimization_barrier(emb_weight.reshape(BUCKETS // PACK, MDIM))
    emb_rm = emb_packed.reshape(BUCKETS, EDIM)

    h = _GATHER(tok, prev, emb_rm)
    h_packed = h.reshape(TOTAL // PACK, MDIM)

    pt = proj_weight.T * scale
    p4 = jnp.zeros((MDIM, PACK * MDIM), jnp.float32)
    for c in range(PACK):
        p4 = p4.at[c * EDIM:(c + 1) * EDIM, c * MDIM:(c + 1) * MDIM].set(pt)

    out_packed = _project(h_packed, p4)
    return out_packed.reshape(B, SEQ, MDIM)


# speedup vs baseline: 22.3326x; 1.3252x over previous
import functools

import jax
import jax.numpy as jnp
from jax import lax
from jax.experimental import pallas as pl
from jax.experimental.pallas import tpu as pltpu
from jax.experimental.pallas import tpu_sc as plsc

BUCKETS = 1000000
MOD = BUCKETS - 1
EDIM = 32
MDIM = 128
B = 4096
SEQ = 200
TOTAL = B * SEQ

LANES = 16
NW = 32
PER_W = TOTAL // NW
CHUNK = 2560
NCHUNK = PER_W // CHUNK
GSZ = 128
NG = CHUNK // GSZ


def _sc_hash_gather():
    mesh = plsc.VectorSubcoreMesh(core_axis_name="c", subcore_axis_name="s")

    @functools.partial(
        pl.kernel,
        mesh=mesh,
        compiler_params=pltpu.CompilerParams(use_tc_tiling_on_sc=False),
        out_type=jax.ShapeDtypeStruct((TOTAL, EDIM), jnp.float32),
        scratch_types=[
            pltpu.VMEM((CHUNK,), jnp.int32),
            pltpu.VMEM((CHUNK,), jnp.int32),
            pltpu.VMEM((CHUNK,), jnp.int32),
            pltpu.VMEM((CHUNK, EDIM), jnp.float32),
            pltpu.SemaphoreType.DMA,
        ],
    )
    def body(cur_hbm, prev_hbm, emb_hbm, out_hbm, cur_v, prev_v, idx_v, rows_v, sem):
        wid = lax.axis_index("s") * 2 + lax.axis_index("c")
        wbase = wid * PER_W

        def chunk_body(s, carry):
            base = wbase + s * CHUNK
            pltpu.sync_copy(cur_hbm.at[pl.ds(base, CHUNK)], cur_v)
            pltpu.sync_copy(prev_hbm.at[pl.ds(base, CHUNK)], prev_v)

            def hash_body(i, c2):
                off = i * LANES
                t1 = cur_v[pl.ds(off, LANES)]
                t0 = prev_v[pl.ds(off, LANES)]
                a = t1 * jnp.int32(36313)
                bb = t0 * jnp.int32(27191)
                r = lax.rem(lax.bitwise_xor(a, bb), jnp.int32(MOD))
                r = jnp.where(r < 0, r + jnp.int32(MOD), r)
                p = base + off + lax.iota(jnp.int32, LANES)
                first = lax.rem(p, jnp.int32(SEQ)) == 0
                idx_v[pl.ds(off, LANES)] = jnp.where(first, jnp.int32(MOD), r)
                return c2

            lax.fori_loop(0, CHUNK // LANES, hash_body, 0)

            copies = []
            for g in range(NG):
                copies.append(
                    pltpu.async_copy(
                        emb_hbm.at[idx_v.at[pl.ds(g * GSZ, GSZ)]],
                        rows_v.at[pl.ds(g * GSZ, GSZ)],
                        sem,
                    )
                )
            for cp in copies:
                cp.wait()
            pltpu.sync_copy(rows_v, out_hbm.at[pl.ds(base, CHUNK)])
            return carry

        lax.fori_loop(0, NCHUNK, chunk_body, 0)

    return body


_GATHER = _sc_hash_gather()

PACK = MDIM // EDIM
BBLK = 16


def _proj_body(h_ref, p_ref, o_ref):
    r = lax.dot_general(
        h_ref[...], p_ref[...], (((1,), (0,)), ((), ())),
        preferred_element_type=jnp.float32,
    )
    n_tok = BBLK * SEQ
    o_ref[...] = r.reshape(n_tok, MDIM).reshape(BBLK, SEQ, MDIM)


def _project(h_packed, p4):
    rows_blk = BBLK * SEQ // PACK
    grid = (B // BBLK,)
    return pl.pallas_call(
        _proj_body,
        grid=grid,
        in_specs=[
            pl.BlockSpec((rows_blk, MDIM), lambda i: (i, 0)),
            pl.BlockSpec((MDIM, PACK * MDIM), lambda i: (0, 0)),
        ],
        out_specs=pl.BlockSpec((BBLK, SEQ, MDIM), lambda i: (i, 0, 0)),
        out_shape=jax.ShapeDtypeStruct((B, SEQ, MDIM), jnp.float32),
    )(h_packed, p4)


def kernel(token_ids, emb_weight, proj_weight, scale):
    tok = token_ids.reshape(-1).astype(jnp.int32)
    prev = jnp.concatenate([tok[:1], tok[:-1]])

    h = _GATHER(tok, prev, emb_weight)
    h_packed = h.reshape(TOTAL // PACK, MDIM)

    pt = proj_weight.T * scale
    p4 = jnp.zeros((MDIM, PACK * MDIM), jnp.float32)
    for c in range(PACK):
        p4 = p4.at[c * EDIM:(c + 1) * EDIM, c * MDIM:(c + 1) * MDIM].set(pt)

    return _project(h_packed, p4)


# speedup vs baseline: 29.6180x; 1.3262x over previous
import functools

import jax
import jax.numpy as jnp
from jax import lax
from jax.experimental import pallas as pl
from jax.experimental.pallas import tpu as pltpu
from jax.experimental.pallas import tpu_sc as plsc

BUCKETS = 1000000
MOD = BUCKETS - 1
EDIM = 32
MDIM = 128
B = 4096
SEQ = 200
TOTAL = B * SEQ

LANES = 16
NW = 32
PER_W = TOTAL // NW
CHUNK = 2560
NCHUNK = PER_W // CHUNK
GSZ = 128
NG = CHUNK // GSZ

KB = 8192
NPBLK = -(-BUCKETS // KB)
PROWS = NPBLK * (KB // 4)


def _sc_hash_gather():
    mesh = plsc.VectorSubcoreMesh(core_axis_name="c", subcore_axis_name="s")

    @functools.partial(
        pl.kernel,
        mesh=mesh,
        compiler_params=pltpu.CompilerParams(use_tc_tiling_on_sc=False),
        out_type=jax.ShapeDtypeStruct((TOTAL, EDIM), jnp.float32),
        scratch_types=[
            pltpu.VMEM((CHUNK,), jnp.int32),
            pltpu.VMEM((CHUNK,), jnp.int32),
            pltpu.VMEM((CHUNK,), jnp.int32),
            pltpu.VMEM((CHUNK, EDIM), jnp.float32),
            pltpu.SemaphoreType.DMA,
        ],
    )
    def body(cur_hbm, prev_hbm, emb_hbm, out_hbm, cur_v, prev_v, idx_v, rows_v, sem):
        wid = lax.axis_index("s") * 2 + lax.axis_index("c")
        wbase = wid * PER_W

        def chunk_body(s, carry):
            base = wbase + s * CHUNK
            pltpu.sync_copy(cur_hbm.at[pl.ds(base, CHUNK)], cur_v)
            pltpu.sync_copy(prev_hbm.at[pl.ds(base, CHUNK)], prev_v)

            def hash_body(i, c2):
                off = i * LANES
                t1 = cur_v[pl.ds(off, LANES)]
                t0 = prev_v[pl.ds(off, LANES)]
                a = t1 * jnp.int32(36313)
                bb = t0 * jnp.int32(27191)
                r = lax.rem(lax.bitwise_xor(a, bb), jnp.int32(MOD))
                r = jnp.where(r < 0, r + jnp.int32(MOD), r)
                p = base + off + lax.iota(jnp.int32, LANES)
                first = lax.rem(p, jnp.int32(SEQ)) == 0
                r = jnp.where(first, jnp.int32(MOD), r)
                j = lax.bitwise_and(r, jnp.int32(KB - 1))
                qd = (r - j) + lax.shift_left(
                    lax.bitwise_and(j, jnp.int32(KB // 4 - 1)), 2
                ) + lax.shift_right_logical(j, 11)
                idx_v[pl.ds(off, LANES)] = qd
                return c2

            lax.fori_loop(0, CHUNK // LANES, hash_body, 0)

            copies = []
            for g in range(NG):
                copies.append(
                    pltpu.async_copy(
                        emb_hbm.at[idx_v.at[pl.ds(g * GSZ, GSZ)]],
                        rows_v.at[pl.ds(g * GSZ, GSZ)],
                        sem,
                    )
                )
            for cp in copies:
                cp.wait()
            pltpu.sync_copy(rows_v, out_hbm.at[pl.ds(base, CHUNK)])
            return carry

        lax.fori_loop(0, NCHUNK, chunk_body, 0)

    return body


_GATHER = _sc_hash_gather()

def _pack_body(t_ref, o_ref):
    t = t_ref[...]
    r = jnp.transpose(t, (1, 0))
    q = KB // 4
    o_ref[...] = jnp.concatenate(
        [r[0:q], r[q:2 * q], r[2 * q:3 * q], r[3 * q:4 * q]], axis=1)


def _pack_table(emb_t):
    return pl.pallas_call(
        _pack_body,
        grid=(NPBLK,),
        in_specs=[pl.BlockSpec((EDIM, KB), lambda i: (0, i))],
        out_specs=pl.BlockSpec((KB // 4, 128), lambda i: (i, 0)),
        out_shape=jax.ShapeDtypeStruct((PROWS, 128), jnp.float32),
    )(emb_t)


PACK = MDIM // EDIM
BBLK = 16


def _proj_body(h_ref, p_ref, o_ref):
    r = lax.dot_general(
        h_ref[...], p_ref[...], (((1,), (0,)), ((), ())),
        preferred_element_type=jnp.float32,
    )
    n_tok = BBLK * SEQ
    o_ref[...] = r.reshape(n_tok, MDIM).reshape(BBLK, SEQ, MDIM)


def _project(h_packed, p4):
    rows_blk = BBLK * SEQ // PACK
    grid = (B // BBLK,)
    return pl.pallas_call(
        _proj_body,
        grid=grid,
        in_specs=[
            pl.BlockSpec((rows_blk, MDIM), lambda i: (i, 0)),
            pl.BlockSpec((MDIM, PACK * MDIM), lambda i: (0, 0)),
        ],
        out_specs=pl.BlockSpec((BBLK, SEQ, MDIM), lambda i: (i, 0, 0)),
        out_shape=jax.ShapeDtypeStruct((B, SEQ, MDIM), jnp.float32),
    )(h_packed, p4)


def kernel(token_ids, emb_weight, proj_weight, scale):
    tok = token_ids.reshape(-1).astype(jnp.int32)
    prev = jnp.concatenate([tok[:1], tok[:-1]])

    emb_rm = _pack_table(emb_weight.T).reshape(PROWS * PACK, EDIM)
    h = _GATHER(tok, prev, emb_rm)
    h_packed = h.reshape(TOTAL // PACK, MDIM)

    pt = proj_weight.T * scale
    p4 = jnp.zeros((MDIM, PACK * MDIM), jnp.float32)
    for c in range(PACK):
        p4 = p4.at[c * EDIM:(c + 1) * EDIM, c * MDIM:(c + 1) * MDIM].set(pt)

    return _project(h_packed, p4)


# speedup vs baseline: 34.7747x; 1.1741x over previous
import functools

import jax
import jax.numpy as jnp
from jax import lax
from jax.experimental import pallas as pl
from jax.experimental.pallas import tpu as pltpu
from jax.experimental.pallas import tpu_sc as plsc

BUCKETS = 1000000
MOD = BUCKETS - 1
EDIM = 32
MDIM = 128
B = 4096
SEQ = 200
TOTAL = B * SEQ

LANES = 16
NW = 32
PER_W = TOTAL // NW
CHUNK = 2560
NCHUNK = PER_W // CHUNK
GSZ = 128
NG = CHUNK // GSZ

KB = 32768
QSHIFT = (KB // 4).bit_length() - 1
NPBLK = -(-BUCKETS // KB)
PROWS = NPBLK * (KB // 4)


def _sc_hash_gather():
    mesh = plsc.VectorSubcoreMesh(core_axis_name="c", subcore_axis_name="s")

    @functools.partial(
        pl.kernel,
        mesh=mesh,
        compiler_params=pltpu.CompilerParams(use_tc_tiling_on_sc=False),
        out_type=jax.ShapeDtypeStruct((TOTAL, EDIM), jnp.float32),
        scratch_types=[
            pltpu.VMEM((CHUNK,), jnp.int32),
            pltpu.VMEM((CHUNK,), jnp.int32),
            pltpu.VMEM((CHUNK,), jnp.int32),
            pltpu.VMEM((CHUNK, EDIM), jnp.float32),
            pltpu.SemaphoreType.DMA,
        ],
    )
    def body(cur_hbm, prev_hbm, emb_hbm, out_hbm, cur_v, prev_v, idx_v, rows_v, sem):
        wid = lax.axis_index("s") * 2 + lax.axis_index("c")
        wbase = wid * PER_W

        def chunk_body(s, carry):
            base = wbase + s * CHUNK
            pltpu.sync_copy(cur_hbm.at[pl.ds(base, CHUNK)], cur_v)
            pltpu.sync_copy(prev_hbm.at[pl.ds(base, CHUNK)], prev_v)

            def hash_body(i, c2):
                off = i * LANES
                t1 = cur_v[pl.ds(off, LANES)]
                t0 = prev_v[pl.ds(off, LANES)]
                a = t1 * jnp.int32(36313)
                bb = t0 * jnp.int32(27191)
                r = lax.rem(lax.bitwise_xor(a, bb), jnp.int32(MOD))
                r = jnp.where(r < 0, r + jnp.int32(MOD), r)
                p = base + off + lax.iota(jnp.int32, LANES)
                first = lax.rem(p, jnp.int32(SEQ)) == 0
                r = jnp.where(first, jnp.int32(MOD), r)
                j = lax.bitwise_and(r, jnp.int32(KB - 1))
                qd = (r - j) + lax.shift_left(
                    lax.bitwise_and(j, jnp.int32(KB // 4 - 1)), 2
                ) + lax.shift_right_logical(j, QSHIFT)
                idx_v[pl.ds(off, LANES)] = qd
                return c2

            lax.fori_loop(0, CHUNK // LANES, hash_body, 0)

            copies = []
            for g in range(NG):
                copies.append(
                    pltpu.async_copy(
                        emb_hbm.at[idx_v.at[pl.ds(g * GSZ, GSZ)]],
                        rows_v.at[pl.ds(g * GSZ, GSZ)],
                        sem,
                    )
                )
            for cp in copies:
                cp.wait()
            pltpu.sync_copy(rows_v, out_hbm.at[pl.ds(base, CHUNK)])
            return carry

        lax.fori_loop(0, NCHUNK, chunk_body, 0)

    return body


_GATHER = _sc_hash_gather()

def _pack_body(t_ref, o_ref):
    t = t_ref[...]
    r = jnp.transpose(t, (1, 0))
    q = KB // 4
    o_ref[:, 0:32] = r[0:q]
    o_ref[:, 32:64] = r[q:2 * q]
    o_ref[:, 64:96] = r[2 * q:3 * q]
    o_ref[:, 96:128] = r[3 * q:4 * q]


def _pack_table(emb_t):
    return pl.pallas_call(
        _pack_body,
        grid=(NPBLK,),
        in_specs=[pl.BlockSpec((EDIM, KB), lambda i: (0, i))],
        out_specs=pl.BlockSpec((KB // 4, 128), lambda i: (i, 0)),
        out_shape=jax.ShapeDtypeStruct((PROWS, 128), jnp.float32),
    )(emb_t)


PACK = MDIM // EDIM
BBLK = 64


def _proj_body(h_ref, p_ref, o_ref):
    r = lax.dot_general(
        h_ref[...], p_ref[...], (((1,), (0,)), ((), ())),
        preferred_element_type=jnp.float32,
    )
    o_ref[...] = r.reshape(BBLK * SEQ, MDIM)


def _project(h_packed, p4):
    rows_blk = BBLK * SEQ // PACK
    grid = (B // BBLK,)
    return pl.pallas_call(
        _proj_body,
        grid=grid,
        in_specs=[
            pl.BlockSpec((rows_blk, MDIM), lambda i: (i, 0)),
            pl.BlockSpec((MDIM, PACK * MDIM), lambda i: (0, 0)),
        ],
        out_specs=pl.BlockSpec((BBLK * SEQ, MDIM), lambda i: (i, 0)),
        out_shape=jax.ShapeDtypeStruct((TOTAL, MDIM), jnp.float32),
    )(h_packed, p4)


def kernel(token_ids, emb_weight, proj_weight, scale):
    tok = token_ids.reshape(-1).astype(jnp.int32)
    prev = jnp.concatenate([tok[:1], tok[:-1]])

    emb_rm = _pack_table(emb_weight.T).reshape(PROWS * PACK, EDIM)
    h = _GATHER(tok, prev, emb_rm)
    h_packed = h.reshape(TOTAL // PACK, MDIM)

    pt = proj_weight.T * scale
    p4 = jnp.zeros((MDIM, PACK * MDIM), jnp.float32)
    for c in range(PACK):
        p4 = p4.at[c * EDIM:(c + 1) * EDIM, c * MDIM:(c + 1) * MDIM].set(pt)

    return _project(h_packed, p4).reshape(B, SEQ, MDIM)


# speedup vs baseline: 36.0141x; 1.0356x over previous
import functools

import jax
import jax.numpy as jnp
from jax import lax
from jax.experimental import pallas as pl
from jax.experimental.pallas import tpu as pltpu
from jax.experimental.pallas import tpu_sc as plsc

BUCKETS = 1000000
MOD = BUCKETS - 1
EDIM = 32
MDIM = 128
B = 4096
SEQ = 200
TOTAL = B * SEQ

LANES = 16
NW = 32
PER_W = TOTAL // NW
CHUNK = 1280
NCHUNK = PER_W // CHUNK
GSZ = 128
NG = CHUNK // GSZ

KB = 32768
QSHIFT = (KB // 4).bit_length() - 1
NPBLK = -(-BUCKETS // KB)
PROWS = NPBLK * (KB // 4)


def _sc_hash():
    mesh = plsc.VectorSubcoreMesh(core_axis_name="c", subcore_axis_name="s")

    @functools.partial(
        pl.kernel,
        mesh=mesh,
        compiler_params=pltpu.CompilerParams(use_tc_tiling_on_sc=False),
        out_type=jax.ShapeDtypeStruct((TOTAL,), jnp.int32),
        scratch_types=[
            pltpu.VMEM((PER_W,), jnp.int32),
            pltpu.VMEM((PER_W,), jnp.int32),
            pltpu.VMEM((PER_W,), jnp.int32),
        ],
    )
    def body(cur_hbm, prev_hbm, idx_hbm, cur_v, prev_v, idx_v):
        wid = lax.axis_index("s") * 2 + lax.axis_index("c")
        wbase = wid * PER_W
        pltpu.sync_copy(cur_hbm.at[pl.ds(wbase, PER_W)], cur_v)
        pltpu.sync_copy(prev_hbm.at[pl.ds(wbase, PER_W)], prev_v)

        def hash_body(i, c2):
            off = i * LANES
            t1 = cur_v[pl.ds(off, LANES)]
            t0 = prev_v[pl.ds(off, LANES)]
            a = t1 * jnp.int32(36313)
            bb = t0 * jnp.int32(27191)
            r = lax.rem(lax.bitwise_xor(a, bb), jnp.int32(MOD))
            r = jnp.where(r < 0, r + jnp.int32(MOD), r)
            p = wbase + off + lax.iota(jnp.int32, LANES)
            first = lax.rem(p, jnp.int32(SEQ)) == 0
            r = jnp.where(first, jnp.int32(MOD), r)
            j = lax.bitwise_and(r, jnp.int32(KB - 1))
            qd = (r - j) + lax.shift_left(
                lax.bitwise_and(j, jnp.int32(KB // 4 - 1)), 2
            ) + lax.shift_right_logical(j, QSHIFT)
            idx_v[pl.ds(off, LANES)] = qd
            return c2

        lax.fori_loop(0, PER_W // LANES, hash_body, 0)
        pltpu.sync_copy(idx_v, idx_hbm.at[pl.ds(wbase, PER_W)])

    return body


def _sc_gather():
    mesh = plsc.VectorSubcoreMesh(core_axis_name="c", subcore_axis_name="s")

    @functools.partial(
        pl.kernel,
        mesh=mesh,
        compiler_params=pltpu.CompilerParams(use_tc_tiling_on_sc=False),
        out_type=jax.ShapeDtypeStruct((TOTAL, EDIM), jnp.float32),
        scratch_types=[
            pltpu.VMEM((PER_W,), jnp.int32),
            pltpu.VMEM((2, CHUNK, EDIM), jnp.float32),
            pltpu.SemaphoreType.DMA((2,)),
            pltpu.SemaphoreType.DMA((2,)),
        ],
    )
    def body(idx_hbm, emb_hbm, out_hbm, idx_v, rows_v, gsem, osem):
        wid = lax.axis_index("s") * 2 + lax.axis_index("c")
        wbase = wid * PER_W
        pltpu.sync_copy(idx_hbm.at[pl.ds(wbase, PER_W)], idx_v)

        def fire(s, slot):
            for g in range(NG):
                pltpu.async_copy(
                    emb_hbm.at[idx_v.at[pl.ds(s * CHUNK + g * GSZ, GSZ)]],
                    rows_v.at[slot, pl.ds(g * GSZ, GSZ)],
                    gsem.at[slot],
                )

        def drain(sem_arr, slot):
            pltpu.make_async_copy(
                emb_hbm.at[pl.ds(0, CHUNK)], rows_v.at[slot], sem_arr.at[slot]
            ).wait()

        fire(0, 0)

        def chunk_body(s, carry):
            slot = lax.rem(s, 2)
            drain(gsem, slot)

            @pl.when(s + 1 < NCHUNK)
            def _():
                @pl.when(s >= 1)
                def _():
                    drain(osem, 1 - slot)
                fire(s + 1, 1 - slot)

            pltpu.async_copy(
                rows_v.at[slot],
                out_hbm.at[pl.ds(wbase + s * CHUNK, CHUNK)],
                osem.at[slot],
            )
            return carry

        lax.fori_loop(0, NCHUNK, chunk_body, 0)
        drain(osem, 0)
        drain(osem, 1)

    return body


_HASH = _sc_hash()
_GATHER = _sc_gather()

def _pack_body(t_ref, o_ref):
    t = t_ref[...]
    r = jnp.transpose(t, (1, 0))
    q = KB // 4
    o_ref[:, 0:32] = r[0:q]
    o_ref[:, 32:64] = r[q:2 * q]
    o_ref[:, 64:96] = r[2 * q:3 * q]
    o_ref[:, 96:128] = r[3 * q:4 * q]


def _pack_table(emb_t):
    return pl.pallas_call(
        _pack_body,
        grid=(NPBLK,),
        in_specs=[pl.BlockSpec((EDIM, KB), lambda i: (0, i))],
        out_specs=pl.BlockSpec((KB // 4, 128), lambda i: (i, 0)),
        out_shape=jax.ShapeDtypeStruct((PROWS, 128), jnp.float32),
    )(emb_t)


PACK = MDIM // EDIM
BBLK = 64


def _proj_body(h_ref, p_ref, o_ref):
    r = lax.dot_general(
        h_ref[...], p_ref[...], (((1,), (0,)), ((), ())),
        preferred_element_type=jnp.float32,
    )
    o_ref[...] = r.reshape(BBLK * SEQ, MDIM)


def _project(h_packed, p4):
    rows_blk = BBLK * SEQ // PACK
    grid = (B // BBLK,)
    return pl.pallas_call(
        _proj_body,
        grid=grid,
        in_specs=[
            pl.BlockSpec((rows_blk, MDIM), lambda i: (i, 0)),
            pl.BlockSpec((MDIM, PACK * MDIM), lambda i: (0, 0)),
        ],
        out_specs=pl.BlockSpec((BBLK * SEQ, MDIM), lambda i: (i, 0)),
        out_shape=jax.ShapeDtypeStruct((TOTAL, MDIM), jnp.float32),
    )(h_packed, p4)


def kernel(token_ids, emb_weight, proj_weight, scale):
    tok = token_ids.reshape(-1).astype(jnp.int32)
    prev = jnp.concatenate([tok[:1], tok[:-1]])

    idx = _HASH(tok, prev)
    emb_rm = _pack_table(emb_weight.T).reshape(PROWS * PACK, EDIM)
    h = _GATHER(idx, emb_rm)
    h_packed = h.reshape(TOTAL // PACK, MDIM)

    pt = proj_weight.T * scale
    p4 = jnp.zeros((MDIM, PACK * MDIM), jnp.float32)
    for c in range(PACK):
        p4 = p4.at[c * EDIM:(c + 1) * EDIM, c * MDIM:(c + 1) * MDIM].set(pt)

    return _project(h_packed, p4).reshape(B, SEQ, MDIM)
